# baseline (speedup 1.0000x reference)
import functools

import jax
import jax.numpy as jnp
from jax import lax
from jax.experimental import pallas as pl
from jax.experimental.pallas import tpu as pltpu
from jax.experimental.pallas import tpu_sc as plsc

H, W = 512, 512
HW = H * W
B = 128
NC, NS, LANES = 2, 16, 16
NW = NC * NS
SPW = B // NW
CHUNK = 32768
NCHUNK = HW // CHUNK
UNROLL = 8
INT_MAX = 2**31 - 1


def _rotreduce(v, tmp, op):
    for shift in (8, 4, 2, 1):
        tmp[pl.ds(0, LANES)] = v
        tmp[pl.ds(LANES, LANES)] = v
        v = op(v, tmp[pl.ds(shift, LANES)])
    return v


def _argmax_gather_kernel(inp, gx, gy, outx, outy, buf0, buf1, bufr, idxv,
                          gatv, rowv, tmpf, tmpi, sem0, sem1, semr, gsem):
    cid = lax.axis_index("c")
    sid = lax.axis_index("s")
    wid = sid * NC + cid
    iota = lax.iota(jnp.int32, LANES)
    bufs = (buf0, buf1)
    sems = (sem0, sem1)

    def chunk_copy(g, buf, sem):
        return pltpu.make_async_copy(inp.at[wid * (SPW * NCHUNK) + g], buf, sem)

    chunk_copy(0, bufs[0], sems[0]).start()

    idx_lanes = jnp.zeros((LANES,), jnp.int32)
    total = SPW * NCHUNK
    neg_inf = jnp.full((LANES,), -jnp.inf, jnp.float32)
    big = jnp.full((LANES,), INT_MAX, jnp.int32)
    iotak = [iota + k * LANES for k in range(UNROLL)]
    for s_local in range(SPW):
        gmax = neg_inf
        cbest = jnp.zeros((LANES,), jnp.int32)
        for c in range(NCHUNK):
            g = s_local * NCHUNK + c
            buf, sem = bufs[g % 2], sems[g % 2]
            if g + 1 < total:
                chunk_copy(g + 1, bufs[(g + 1) % 2], sems[(g + 1) % 2]).start()
            chunk_copy(g, buf, sem).wait()

            def body(i, carry, buf=buf):
                return tuple(
                    jnp.maximum(
                        carry[k],
                        buf[pl.ds(i * (UNROLL * LANES) + k * LANES, LANES)])
                    for k in range(UNROLL))

            accl = list(lax.fori_loop(0, CHUNK // (UNROLL * LANES), body,
                                      (neg_inf,) * UNROLL))
            while len(accl) > 1:
                accl = [jnp.maximum(accl[j], accl[j + 1])
                        for j in range(0, len(accl), 2)]
            cm = _rotreduce(accl[0], tmpf, jnp.maximum)
            improved = cm > gmax
            cbest = jnp.where(improved, jnp.full((LANES,), c, jnp.int32), cbest)
            gmax = jnp.where(improved, cm, gmax)

        rowv[...] = cbest + (wid * SPW + s_local) * NCHUNK
        pltpu.make_async_copy(inp.at[rowv.at[pl.ds(0, 1)]], bufr, semr).start()
        pltpu.make_async_copy(inp.at[rowv.at[pl.ds(0, 1)]], bufr, semr).wait()

        def rbody(i, carry):
            ibase = jnp.full((LANES,), i * (UNROLL * LANES), jnp.int32)
            out = []
            for k in range(UNROLL):
                v = bufr[0, pl.ds(i * (UNROLL * LANES) + k * LANES, LANES)]
                out.append(jnp.minimum(
                    carry[k],
                    jnp.where(v == gmax, ibase + iotak[k], big)))
            return tuple(out)

        rmins = list(lax.fori_loop(0, CHUNK // (UNROLL * LANES), rbody,
                                   (big,) * UNROLL))
        while len(rmins) > 1:
            rmins = [jnp.minimum(rmins[j], rmins[j + 1])
                     for j in range(0, len(rmins), 2)]
        rloc = _rotreduce(rmins[0], tmpi, jnp.minimum)
        best = rloc + cbest * CHUNK
        idx_lanes = jnp.where(iota == s_local, best, idx_lanes)

    idxv[...] = idx_lanes
    pltpu.make_async_copy(gx.at[idxv], gatv, gsem).start()
    pltpu.make_async_copy(gx.at[idxv], gatv, gsem).wait()
    pltpu.sync_copy(gatv, outx.at[wid])
    pltpu.make_async_copy(gy.at[idxv], gatv, gsem).start()
    pltpu.make_async_copy(gy.at[idxv], gatv, gsem).wait()
    pltpu.sync_copy(gatv, outy.at[wid])


@jax.jit
def kernel(input, grid_x, grid_y):
    inp2 = input.reshape(B * NCHUNK, CHUNK)
    gx1 = grid_x.reshape(HW)
    gy1 = grid_y.reshape(HW)
    call = functools.partial(
        pl.kernel,
        out_type=[
            jax.ShapeDtypeStruct((NW, LANES), jnp.float32),
            jax.ShapeDtypeStruct((NW, LANES), jnp.float32),
        ],
        mesh=plsc.VectorSubcoreMesh(core_axis_name="c", subcore_axis_name="s"),
        scratch_types=[
            pltpu.VMEM((CHUNK,), jnp.float32),
            pltpu.VMEM((CHUNK,), jnp.float32),
            pltpu.VMEM((1, CHUNK), jnp.float32),
            pltpu.VMEM((LANES,), jnp.int32),
            pltpu.VMEM((LANES,), jnp.float32),
            pltpu.VMEM((LANES,), jnp.int32),
            pltpu.VMEM((2 * LANES,), jnp.float32),
            pltpu.VMEM((2 * LANES,), jnp.int32),
            pltpu.SemaphoreType.DMA,
            pltpu.SemaphoreType.DMA,
            pltpu.SemaphoreType.DMA,
            pltpu.SemaphoreType.DMA,
        ],
    )(_argmax_gather_kernel)
    outx, outy = call(inp2, gx1, gy1)
    x = outx[:, :SPW].reshape(B, 1)
    y = outy[:, :SPW].reshape(B, 1)
    return jnp.concatenate((x, y), axis=1)


# speedup vs baseline: 1.0066x; 1.0066x over previous
import functools

import jax
import jax.numpy as jnp
from jax import lax
from jax.experimental import pallas as pl
from jax.experimental.pallas import tpu as pltpu
from jax.experimental.pallas import tpu_sc as plsc

H, W = 512, 512
HW = H * W
B = 128
NC, NS, LANES = 2, 16, 16
NW = NC * NS
GRP = 8
NG = B // GRP
TCH = 32768
NTCH = HW // TCH
UNROLL_TC = 4
INT_MAX = 2**31 - 1


def _tc_argmax_kernel(x_ref, out_ref, accv, acci):
    j = pl.program_id(1)
    lane = lax.broadcasted_iota(jnp.int32, (GRP, 128), 1)

    @pl.when(j == 0)
    def _():
        accv[...] = jnp.full((GRP, 128), -jnp.inf, jnp.float32)
        acci[...] = jnp.zeros((GRP, 128), jnp.int32)

    av = accv[...]
    ai = acci[...]
    laneoff = [lane + t * 128 for t in range(UNROLL_TC)]
    base = j * TCH

    def body(k, carry):
        av, ai = carry
        kbase = base + k * (UNROLL_TC * 128)
        for t in range(UNROLL_TC):
            v = x_ref[:, pl.ds(k * (UNROLL_TC * 128) + t * 128, 128)]
            iv = laneoff[t] + kbase
            m = v > av
            av = jnp.where(m, v, av)
            ai = jnp.where(m, iv, ai)
        return av, ai

    av, ai = lax.fori_loop(0, TCH // (UNROLL_TC * 128), body, (av, ai))
    accv[...] = av
    acci[...] = ai

    @pl.when(j == NTCH - 1)
    def _():
        m = jnp.max(av, axis=1, keepdims=True)
        cand = jnp.where(av == m, ai, jnp.int32(INT_MAX))
        out_ref[...] = jnp.min(cand, axis=1, keepdims=True)


def _sc_gather_kernel(idx_hbm, gx, gy, outx, outy, idxv, gatv, sem):
    cid = lax.axis_index("c")
    sid = lax.axis_index("s")
    wid = sid * NC + cid

    @pl.when(wid == 0)
    def _():
        pltpu.sync_copy(idx_hbm, idxv)
        pltpu.make_async_copy(gx.at[idxv], gatv, sem).start()
        pltpu.make_async_copy(gx.at[idxv], gatv, sem).wait()
        pltpu.sync_copy(gatv, outx)
        pltpu.make_async_copy(gy.at[idxv], gatv, sem).start()
        pltpu.make_async_copy(gy.at[idxv], gatv, sem).wait()
        pltpu.sync_copy(gatv, outy)


@jax.jit
def kernel(input, grid_x, grid_y):
    xr = input.reshape(B, HW)
    gx1 = grid_x.reshape(HW)
    gy1 = grid_y.reshape(HW)

    idx = pl.pallas_call(
        _tc_argmax_kernel,
        out_shape=jax.ShapeDtypeStruct((B, 1), jnp.int32),
        grid=(NG, NTCH),
        in_specs=[pl.BlockSpec((GRP, TCH), lambda i, j: (i, j))],
        out_specs=pl.BlockSpec((GRP, 1), lambda i, j: (i, 0)),
        scratch_shapes=[
            pltpu.VMEM((GRP, 128), jnp.float32),
            pltpu.VMEM((GRP, 128), jnp.int32),
        ],
    )(xr)

    gather = functools.partial(
        pl.kernel,
        out_type=[
            jax.ShapeDtypeStruct((B,), jnp.float32),
            jax.ShapeDtypeStruct((B,), jnp.float32),
        ],
        mesh=plsc.VectorSubcoreMesh(core_axis_name="c", subcore_axis_name="s"),
        scratch_types=[
            pltpu.VMEM((B,), jnp.int32),
            pltpu.VMEM((B,), jnp.float32),
            pltpu.SemaphoreType.DMA,
        ],
    )(_sc_gather_kernel)
    x, y = gather(idx.reshape(B), gx1, gy1)
    return jnp.concatenate((x.reshape(B, 1), y.reshape(B, 1)), axis=1)


# speedup vs baseline: 1.6108x; 1.6003x over previous
import functools

import jax
import jax.numpy as jnp
from jax import lax
from jax.experimental import pallas as pl
from jax.experimental.pallas import tpu as pltpu
from jax.experimental.pallas import tpu_sc as plsc

H, W = 512, 512
HW = H * W
B = 128
NC, NS, LANES = 2, 16, 16
NW = NC * NS
GRP = 8
RBLK = 64
NG = B // GRP
NTCH = H // RBLK
INT_MAX = 2**31 - 1


def _tc_argmax_kernel(x_ref, out_ref):
    pre = (lax.broadcasted_iota(jnp.int32, (8, W), 0) * W
           + lax.broadcasted_iota(jnp.int32, (8, W), 1))
    neg = jnp.full((8, W), -jnp.inf, jnp.float32)
    zer = jnp.zeros((8, W), jnp.int32)

    def body(k, carry):
        av, ai = carry
        v = x_ref[0, pl.ds(8 * k, 8), :]
        iv = pre + 8 * k * W
        m = v > av
        return jnp.where(m, v, av), jnp.where(m, iv, ai)

    av, ai = lax.fori_loop(0, H // 8, body, (neg, zer))
    m = jnp.max(av)
    cand = jnp.where(av == m, ai, jnp.int32(INT_MAX))
    out_ref[...] = jnp.broadcast_to(jnp.min(cand), (1, 1, 128))


def _sc_gather_kernel(idx_hbm, gx, gy, outx, outy, idxv, gatv, sem):
    cid = lax.axis_index("c")
    sid = lax.axis_index("s")
    wid = sid * NC + cid

    @pl.when(wid == 0)
    def _():
        pltpu.sync_copy(idx_hbm, idxv)
        pltpu.make_async_copy(gx.at[idxv], gatv, sem).start()
        pltpu.make_async_copy(gx.at[idxv], gatv, sem).wait()
        pltpu.sync_copy(gatv, outx)
        pltpu.make_async_copy(gy.at[idxv], gatv, sem).start()
        pltpu.make_async_copy(gy.at[idxv], gatv, sem).wait()
        pltpu.sync_copy(gatv, outy)


@jax.jit
def kernel(input, grid_x, grid_y):
    xr = input.reshape(B, H, W)
    gx1 = grid_x.reshape(HW)
    gy1 = grid_y.reshape(HW)

    idx3 = pl.pallas_call(
        _tc_argmax_kernel,
        out_shape=jax.ShapeDtypeStruct((B, 1, 128), jnp.int32),
        grid=(B,),
        in_specs=[pl.BlockSpec((1, H, W), lambda i: (i, 0, 0))],
        out_specs=pl.BlockSpec((1, 1, 128), lambda i: (i, 0, 0)),
    )(xr)
    idx = idx3[:, 0, 0]

    gather = functools.partial(
        pl.kernel,
        out_type=[
            jax.ShapeDtypeStruct((B,), jnp.float32),
            jax.ShapeDtypeStruct((B,), jnp.float32),
        ],
        mesh=plsc.VectorSubcoreMesh(core_axis_name="c", subcore_axis_name="s"),
        scratch_types=[
            pltpu.VMEM((B,), jnp.int32),
            pltpu.VMEM((B,), jnp.float32),
            pltpu.SemaphoreType.DMA,
        ],
    )(_sc_gather_kernel)
    x, y = gather(idx, gx1, gy1)
    return jnp.concatenate((x.reshape(B, 1), y.reshape(B, 1)), axis=1)


# speedup vs baseline: 1.7568x; 1.0906x over previous
import functools

import jax
import jax.numpy as jnp
from jax import lax
from jax.experimental import pallas as pl
from jax.experimental.pallas import tpu as pltpu
from jax.experimental.pallas import tpu_sc as plsc

H, W = 512, 512
HW = H * W
B = 128
NC, NS, LANES = 2, 16, 16
NW = NC * NS
UNR = 8
NACC = 4
INT_MAX = 2**31 - 1


def _tc_argmax_kernel(x_ref, out_ref):
    pre = (lax.broadcasted_iota(jnp.int32, (8, W), 0) * W
           + lax.broadcasted_iota(jnp.int32, (8, W), 1))
    neg = jnp.full((8, W), -jnp.inf, jnp.float32)
    zer = jnp.zeros((8, W), jnp.int32)
    ngrp = H // 8

    def body(k, carry):
        acc = list(carry)
        for t in range(UNR):
            kt = k * UNR + t
            v = x_ref[0, pl.ds(kt * 8, 8), :]
            p = t % NACC
            av, ai = acc[2 * p], acc[2 * p + 1]
            m = v > av
            acc[2 * p] = jnp.where(m, v, av)
            acc[2 * p + 1] = jnp.where(m, jnp.full((8, W), kt, jnp.int32), ai)
        return tuple(acc)

    acc = list(lax.fori_loop(0, ngrp // UNR, body, (neg, zer) * NACC))
    pairs = [(acc[2 * p], acc[2 * p + 1] * (8 * W) + pre)
             for p in range(NACC)]
    while len(pairs) > 1:
        out = []
        for q in range(0, len(pairs), 2):
            (av0, ai0), (av1, ai1) = pairs[q], pairs[q + 1]
            better = (av1 > av0) | ((av1 == av0) & (ai1 < ai0))
            out.append((jnp.where(better, av1, av0),
                        jnp.where(better, ai1, ai0)))
        pairs = out
    av, ai = pairs[0]
    m = jnp.max(av)
    cand = jnp.where(av == m, ai, jnp.int32(INT_MAX))
    out_ref[...] = jnp.broadcast_to(jnp.min(cand), (1, 1, 128))


def _sc_gather_kernel(idx_hbm, gx, gy, outx, outy, idxv, gatv, sem):
    cid = lax.axis_index("c")
    sid = lax.axis_index("s")
    wid = sid * NC + cid

    @pl.when(wid == 0)
    def _():
        pltpu.sync_copy(idx_hbm, idxv)
        pltpu.make_async_copy(gx.at[idxv], gatv, sem).start()
        pltpu.make_async_copy(gx.at[idxv], gatv, sem).wait()
        pltpu.sync_copy(gatv, outx)
        pltpu.make_async_copy(gy.at[idxv], gatv, sem).start()
        pltpu.make_async_copy(gy.at[idxv], gatv, sem).wait()
        pltpu.sync_copy(gatv, outy)


@jax.jit
def kernel(input, grid_x, grid_y):
    xr = input.reshape(B, H, W)
    gx1 = grid_x.reshape(HW)
    gy1 = grid_y.reshape(HW)

    idx3 = pl.pallas_call(
        _tc_argmax_kernel,
        out_shape=jax.ShapeDtypeStruct((B, 1, 128), jnp.int32),
        grid=(B,),
        in_specs=[pl.BlockSpec((1, H, W), lambda i: (i, 0, 0))],
        out_specs=pl.BlockSpec((1, 1, 128), lambda i: (i, 0, 0)),
    )(xr)
    idx = idx3[:, 0, 0]

    gather = functools.partial(
        pl.kernel,
        out_type=[
            jax.ShapeDtypeStruct((B,), jnp.float32),
            jax.ShapeDtypeStruct((B,), jnp.float32),
        ],
        mesh=plsc.VectorSubcoreMesh(core_axis_name="c", subcore_axis_name="s"),
        scratch_types=[
            pltpu.VMEM((B,), jnp.int32),
            pltpu.VMEM((B,), jnp.float32),
            pltpu.SemaphoreType.DMA,
        ],
    )(_sc_gather_kernel)
    x, y = gather(idx, gx1, gy1)
    return jnp.concatenate((x.reshape(B, 1), y.reshape(B, 1)), axis=1)


# speedup vs baseline: 2.6357x; 1.5003x over previous
import functools

import jax
import jax.numpy as jnp
from jax import lax
from jax.experimental import pallas as pl
from jax.experimental.pallas import tpu as pltpu
from jax.experimental.pallas import tpu_sc as plsc

H, W = 512, 512
HW = H * W
B = 128
NC, NS, LANES = 2, 16, 16
NW = NC * NS
UNR = 8
NACC = 2
SPB = 4
INT_MAX = 2**31 - 1


def _tc_argmax_kernel(x_ref, out_ref):
    pre = (lax.broadcasted_iota(jnp.int32, (8, W), 0) * W
           + lax.broadcasted_iota(jnp.int32, (8, W), 1))
    neg = jnp.full((8, W), -jnp.inf, jnp.float32)
    zer = jnp.zeros((8, W), jnp.int32)
    ngrp = H // 8

    for g in range(SPB):
        def body(k, carry, g=g):
            acc = list(carry)
            for t in range(UNR):
                kt = k * UNR + t
                v = x_ref[g, pl.ds(kt * 8, 8), :]
                p = t % NACC
                av, ai = acc[2 * p], acc[2 * p + 1]
                m = v > av
                acc[2 * p] = jnp.where(m, v, av)
                acc[2 * p + 1] = jnp.where(
                    m, jnp.full((8, W), kt, jnp.int32), ai)
            return tuple(acc)

        acc = list(lax.fori_loop(0, ngrp // UNR, body, (neg, zer) * NACC))
        pairs = [(acc[2 * p], acc[2 * p + 1] * (8 * W) + pre)
                 for p in range(NACC)]
        while len(pairs) > 1:
            out = []
            for q in range(0, len(pairs), 2):
                (av0, ai0), (av1, ai1) = pairs[q], pairs[q + 1]
                better = (av1 > av0) | ((av1 == av0) & (ai1 < ai0))
                out.append((jnp.where(better, av1, av0),
                            jnp.where(better, ai1, ai0)))
            pairs = out
        av, ai = pairs[0]
        m = jnp.max(av)
        cand = jnp.where(av == m, ai, jnp.int32(INT_MAX))
        out_ref[g] = jnp.broadcast_to(jnp.min(cand), (1, 128))


def _sc_gather_kernel(idx_hbm, gx, gy, outx, outy, idxv, gatv, sem):
    cid = lax.axis_index("c")
    sid = lax.axis_index("s")
    wid = sid * NC + cid

    @pl.when(wid == 0)
    def _():
        pltpu.sync_copy(idx_hbm, idxv)
        pltpu.make_async_copy(gx.at[idxv], gatv, sem).start()
        pltpu.make_async_copy(gx.at[idxv], gatv, sem).wait()
        pltpu.sync_copy(gatv, outx)
        pltpu.make_async_copy(gy.at[idxv], gatv, sem).start()
        pltpu.make_async_copy(gy.at[idxv], gatv, sem).wait()
        pltpu.sync_copy(gatv, outy)


@jax.jit
def kernel(input, grid_x, grid_y):
    xr = input.reshape(B, H, W)
    gx1 = grid_x.reshape(HW)
    gy1 = grid_y.reshape(HW)

    idx3 = pl.pallas_call(
        _tc_argmax_kernel,
        out_shape=jax.ShapeDtypeStruct((B, 1, 128), jnp.int32),
        grid=(B // SPB,),
        in_specs=[pl.BlockSpec((SPB, H, W), lambda i: (i, 0, 0))],
        out_specs=pl.BlockSpec((SPB, 1, 128), lambda i: (i, 0, 0)),
    )(xr)
    idx = idx3[:, 0, 0]

    gather = functools.partial(
        pl.kernel,
        out_type=[
            jax.ShapeDtypeStruct((B,), jnp.float32),
            jax.ShapeDtypeStruct((B,), jnp.float32),
        ],
        mesh=plsc.VectorSubcoreMesh(core_axis_name="c", subcore_axis_name="s"),
        scratch_types=[
            pltpu.VMEM((B,), jnp.int32),
            pltpu.VMEM((B,), jnp.float32),
            pltpu.SemaphoreType.DMA,
        ],
    )(_sc_gather_kernel)
    x, y = gather(idx, gx1, gy1)
    return jnp.concatenate((x.reshape(B, 1), y.reshape(B, 1)), axis=1)


# speedup vs baseline: 2.7104x; 1.0283x over previous
import functools

import jax
import jax.numpy as jnp
from jax import lax
from jax.experimental import pallas as pl
from jax.experimental.pallas import tpu as pltpu
from jax.experimental.pallas import tpu_sc as plsc

H, W = 512, 512
HW = H * W
B = 128
NC, NS, LANES = 2, 16, 16
NW = NC * NS
UNR = 8
NACC = 2
SPB = 8
INT_MAX = 2**31 - 1


def _tc_argmax_kernel(x_ref, out_ref):
    pre = (lax.broadcasted_iota(jnp.int32, (8, W), 0) * W
           + lax.broadcasted_iota(jnp.int32, (8, W), 1))
    neg = jnp.full((8, W), -jnp.inf, jnp.float32)
    zer = jnp.zeros((8, W), jnp.int32)
    ngrp = H // 8

    for g in range(SPB):
        def body(k, carry, g=g):
            acc = list(carry)
            for t in range(UNR):
                kt = k * UNR + t
                v = x_ref[g, pl.ds(kt * 8, 8), :]
                p = t % NACC
                av, ai = acc[2 * p], acc[2 * p + 1]
                m = v > av
                acc[2 * p] = jnp.where(m, v, av)
                acc[2 * p + 1] = jnp.where(
                    m, jnp.full((8, W), kt, jnp.int32), ai)
            return tuple(acc)

        acc = list(lax.fori_loop(0, ngrp // UNR, body, (neg, zer) * NACC))
        pairs = [(acc[2 * p], acc[2 * p + 1] * (8 * W) + pre)
                 for p in range(NACC)]
        while len(pairs) > 1:
            out = []
            for q in range(0, len(pairs), 2):
                (av0, ai0), (av1, ai1) = pairs[q], pairs[q + 1]
                better = (av1 > av0) | ((av1 == av0) & (ai1 < ai0))
                out.append((jnp.where(better, av1, av0),
                            jnp.where(better, ai1, ai0)))
            pairs = out
        av, ai = pairs[0]
        m = jnp.max(av)
        cand = jnp.where(av == m, ai, jnp.int32(INT_MAX))
        out_ref[g] = jnp.broadcast_to(jnp.min(cand), (1, 128))


def _sc_gather_kernel(idx_hbm, gx, gy, outx, outy, idxv, gatv, sem):
    cid = lax.axis_index("c")
    sid = lax.axis_index("s")
    wid = sid * NC + cid

    @pl.when(wid == 0)
    def _():
        pltpu.sync_copy(idx_hbm, idxv)
        pltpu.make_async_copy(gx.at[idxv], gatv, sem).start()
        pltpu.make_async_copy(gx.at[idxv], gatv, sem).wait()
        pltpu.sync_copy(gatv, outx)
        pltpu.make_async_copy(gy.at[idxv], gatv, sem).start()
        pltpu.make_async_copy(gy.at[idxv], gatv, sem).wait()
        pltpu.sync_copy(gatv, outy)


@jax.jit
def kernel(input, grid_x, grid_y):
    xr = input.reshape(B, H, W)
    gx1 = grid_x.reshape(HW)
    gy1 = grid_y.reshape(HW)

    idx3 = pl.pallas_call(
        _tc_argmax_kernel,
        out_shape=jax.ShapeDtypeStruct((B, 1, 128), jnp.int32),
        grid=(B // SPB,),
        in_specs=[pl.BlockSpec((SPB, H, W), lambda i: (i, 0, 0))],
        out_specs=pl.BlockSpec((SPB, 1, 128), lambda i: (i, 0, 0)),
    )(xr)
    idx = idx3[:, 0, 0]

    gather = functools.partial(
        pl.kernel,
        out_type=[
            jax.ShapeDtypeStruct((B,), jnp.float32),
            jax.ShapeDtypeStruct((B,), jnp.float32),
        ],
        mesh=plsc.VectorSubcoreMesh(core_axis_name="c", subcore_axis_name="s"),
        scratch_types=[
            pltpu.VMEM((B,), jnp.int32),
            pltpu.VMEM((B,), jnp.float32),
            pltpu.SemaphoreType.DMA,
        ],
    )(_sc_gather_kernel)
    x, y = gather(idx, gx1, gy1)
    return jnp.concatenate((x.reshape(B, 1), y.reshape(B, 1)), axis=1)
